# baseline (speedup 1.0000x reference)
import jax
import jax.numpy as jnp
from jax import lax
from jax.experimental import pallas as pl
from jax.experimental.pallas import tpu as pltpu
from jax.experimental.pallas import tpu_sc as plsc

NN = 10000
EE = 320000
DD = 128
HH = 128
LL = 6
NC = 2
NS = 16
NW = NC * NS
EPT = EE // NW
CH = 80
NCH = EPT // CH
NP = 10240
F32 = jnp.float32

_MESH = plsc.VectorSubcoreMesh(core_axis_name="c", subcore_axis_name="s")


def _deg_body(dst3_hbm, zeros_hbm, deg_out, idx_v, ones_v, sem0, sem1,
              deg_sh):
    c = lax.axis_index("c")
    s = lax.axis_index("s")
    w = c * NS + s
    z16 = jnp.zeros((16,), F32)
    one_hot = jnp.where(lax.iota(jnp.int32, 16) == 0, 1.0, 0.0).astype(F32)

    @pl.when(s == 0)
    def _():
        pltpu.sync_copy(zeros_hbm, deg_sh)

    @pl.loop(0, CH)
    def _(r):
        for cc in range(1, 8):
            ones_v[r, pl.ds(cc * 16, 16)] = z16
        ones_v[r, pl.ds(0, 16)] = one_hot

    pltpu.sync_copy(dst3_hbm.at[w], idx_v)
    plsc.subcore_barrier()

    @pl.loop(0, NCH - 1, step=2)
    def _(j):
        ca = pltpu.async_copy(ones_v, deg_sh.at[idx_v.at[j]], sem0, add=True)
        cb = pltpu.async_copy(ones_v, deg_sh.at[idx_v.at[j + 1]], sem1,
                              add=True)
        ca.wait()
        cb.wait()

    pltpu.sync_copy(ones_v, deg_sh.at[idx_v.at[NCH - 1]], add=True)

    @pl.when(s == 0)
    def _():
        pltpu.sync_copy(deg_sh, deg_out.at[pl.ds(c * NP, NP)])


_deg_call = pl.kernel(
    _deg_body,
    out_type=jax.ShapeDtypeStruct((NC * NP, 128), F32),
    mesh=_MESH,
    scratch_types=[
        pltpu.VMEM((NCH, CH), jnp.int32),
        pltpu.VMEM((CH, 128), F32),
        pltpu.SemaphoreType.DMA,
        pltpu.SemaphoreType.DMA,
        pltpu.VMEM_SHARED((NP, 128), F32),
    ],
)


def _agg_body(g_hbm, srcf_hbm, dst3_hbm, zeros_hbm, agg_out, src_v, dst_v,
              buf0, buf1, semg0, semg1, sems0, sems1, agg_sh):
    c = lax.axis_index("c")
    s = lax.axis_index("s")
    w = c * NS + s

    @pl.when(s == 0)
    def _():
        pltpu.sync_copy(zeros_hbm, agg_sh)

    pltpu.sync_copy(srcf_hbm.at[pl.ds(w * EPT, EPT)], src_v)
    pltpu.sync_copy(dst3_hbm.at[w], dst_v)
    plsc.subcore_barrier()

    def _gat(j, buf, sem):
        return pltpu.async_copy(g_hbm.at[src_v.at[pl.ds(j * CH, CH)]], buf,
                                sem)

    _gat(0, buf0, semg0)
    _gat(1, buf1, semg1)

    @pl.loop(0, NCH - 3, step=2)
    def _(j):
        pltpu.make_async_copy(g_hbm.at[src_v.at[pl.ds(j * CH, CH)]], buf0,
                              semg0).wait()
        sa = pltpu.async_copy(buf0, agg_sh.at[dst_v.at[j]], sems0, add=True)
        pltpu.make_async_copy(g_hbm.at[src_v.at[pl.ds(j * CH, CH)]], buf1,
                              semg1).wait()
        sb = pltpu.async_copy(buf1, agg_sh.at[dst_v.at[j + 1]], sems1,
                              add=True)
        sa.wait()
        _gat(j + 2, buf0, semg0)
        sb.wait()
        _gat(j + 3, buf1, semg1)

    pltpu.make_async_copy(g_hbm.at[src_v.at[pl.ds(0, CH)]], buf0,
                          semg0).wait()
    pltpu.sync_copy(buf0, agg_sh.at[dst_v.at[NCH - 3]], add=True)
    pltpu.make_async_copy(g_hbm.at[src_v.at[pl.ds(0, CH)]], buf1,
                          semg1).wait()
    pltpu.sync_copy(buf1, agg_sh.at[dst_v.at[NCH - 2]], add=True)
    pltpu.async_copy(g_hbm.at[src_v.at[pl.ds((NCH - 1) * CH, CH)]], buf0,
                     semg0).wait()
    pltpu.sync_copy(buf0, agg_sh.at[dst_v.at[NCH - 1]], add=True)
    plsc.subcore_barrier()

    @pl.when(s == 0)
    def _():
        pltpu.sync_copy(agg_sh, agg_out.at[pl.ds(c * NP, NP)])


_agg_call = pl.kernel(
    _agg_body,
    out_type=jax.ShapeDtypeStruct((NC * NP, HH), F32),
    mesh=_MESH,
    scratch_types=[
        pltpu.VMEM((EPT,), jnp.int32),
        pltpu.VMEM((NCH, CH), jnp.int32),
        pltpu.VMEM((CH, HH), F32),
        pltpu.VMEM((CH, HH), F32),
        pltpu.SemaphoreType.DMA,
        pltpu.SemaphoreType.DMA,
        pltpu.SemaphoreType.DMA,
        pltpu.SemaphoreType.DMA,
        pltpu.VMEM_SHARED((NP, HH), F32),
    ],
)


def _fin_body(ya_hbm, yb_hbm, s2_hbm, d2_hbm, out_hbm, s2_v, d2_v, bufA0,
              bufB0, bufA1, bufB1, obuf0, obuf1, semA0, semB0, semA1, semB1):
    c = lax.axis_index("c")
    s = lax.axis_index("s")
    w = c * NS + s
    pltpu.sync_copy(s2_hbm.at[w], s2_v)
    pltpu.sync_copy(d2_hbm.at[w], d2_v)

    def _chunk(j, bufA, bufB, obuf):
        @pl.loop(0, CH, unroll=8)
        def _(r):
            obuf[r, :] = bufA[r, pl.ds(0, 16)] + bufB[r, pl.ds(0, 16)]

        pltpu.sync_copy(obuf, out_hbm.at[pl.ds(w * EPT + j * CH, CH)])

    @pl.loop(0, NCH - 1, step=2)
    def _(j):
        cA0 = pltpu.async_copy(ya_hbm.at[s2_v.at[j]], bufA0, semA0)
        cB0 = pltpu.async_copy(yb_hbm.at[d2_v.at[j]], bufB0, semB0)
        cA1 = pltpu.async_copy(ya_hbm.at[s2_v.at[j + 1]], bufA1, semA1)
        cB1 = pltpu.async_copy(yb_hbm.at[d2_v.at[j + 1]], bufB1, semB1)
        cA0.wait()
        cB0.wait()
        _chunk(j, bufA0, bufB0, obuf0)
        cA1.wait()
        cB1.wait()
        _chunk(j + 1, bufA1, bufB1, obuf1)

    cA0 = pltpu.async_copy(ya_hbm.at[s2_v.at[NCH - 1]], bufA0, semA0)
    cB0 = pltpu.async_copy(yb_hbm.at[d2_v.at[NCH - 1]], bufB0, semB0)
    cA0.wait()
    cB0.wait()
    _chunk(NCH - 1, bufA0, bufB0, obuf0)


_fin_call = pl.kernel(
    _fin_body,
    out_type=jax.ShapeDtypeStruct((EE, 16), F32),
    mesh=_MESH,
    scratch_types=[
        pltpu.VMEM((NCH, CH), jnp.int32),
        pltpu.VMEM((NCH, CH), jnp.int32),
        pltpu.VMEM((CH, 128), F32),
        pltpu.VMEM((CH, 128), F32),
        pltpu.VMEM((CH, 128), F32),
        pltpu.VMEM((CH, 128), F32),
        pltpu.VMEM((CH, 16), F32),
        pltpu.VMEM((CH, 16), F32),
        pltpu.SemaphoreType.DMA,
        pltpu.SemaphoreType.DMA,
        pltpu.SemaphoreType.DMA,
        pltpu.SemaphoreType.DMA,
    ],
)


def _matT(a, b):
    return lax.dot_general(a, b, (((1,), (1,)), ((), ())),
                           preferred_element_type=F32)


def _embed_body(x_ref, we_ref, be_ref, w1_ref, xe_out, h1_out):
    xe = _matT(x_ref[...], we_ref[...]) + be_ref[...][None, :]
    xe_out[...] = xe
    h1_out[...] = _matT(xe, w1_ref[...])


_embed_call = pl.pallas_call(
    _embed_body,
    out_shape=(
        jax.ShapeDtypeStruct((NN, HH), F32),
        jax.ShapeDtypeStruct((NN, HH), F32),
    ),
)


def _scale_body(h1_ref, deg_ref, dinv_out, g_out):
    deg = deg_ref[0:NN, 0:1] + deg_ref[NP:NP + NN, 0:1] + 1.0
    dinv = lax.rsqrt(deg)
    dinv_out[...] = dinv
    g_out[...] = h1_ref[...] * dinv


_scale_call = pl.pallas_call(
    _scale_body,
    out_shape=(
        jax.ShapeDtypeStruct((NN, 1), F32),
        jax.ShapeDtypeStruct((NN, HH), F32),
    ),
)


def _layer_core(agg_ref, g_ref, dinv, b_ref, gam_ref, bet_ref, xe_ref):
    out = ((agg_ref[0:NN] + agg_ref[NP:NP + NN] + g_ref[...]) * dinv
           + b_ref[...][None, :])
    m = jnp.mean(out, axis=0, keepdims=True)
    d0 = out - m
    v = jnp.mean(d0 * d0, axis=0, keepdims=True)
    hh = d0 * lax.rsqrt(v + 1e-5) * gam_ref[...][None, :] + bet_ref[...][None, :]
    return xe_ref[...] + jnp.maximum(hh, 0.0)


def _layer_body(agg_ref, g_ref, dinv_ref, b_ref, gam_ref, bet_ref, xe_ref,
                wn_ref, xe_out, g_out):
    dinv = dinv_ref[...]
    xe = _layer_core(agg_ref, g_ref, dinv, b_ref, gam_ref, bet_ref, xe_ref)
    xe_out[...] = xe
    g_out[...] = _matT(xe, wn_ref[...]) * dinv


_layer_call = pl.pallas_call(
    _layer_body,
    out_shape=(
        jax.ShapeDtypeStruct((NN, HH), F32),
        jax.ShapeDtypeStruct((NN, HH), F32),
    ),
)


def _last_body(agg_ref, g_ref, dinv_ref, b_ref, gam_ref, bet_ref, xe_ref,
               wfa_ref, wfb_ref, bfc_ref, ya_out, yb_out):
    dinv = dinv_ref[...]
    xe = _layer_core(agg_ref, g_ref, dinv, b_ref, gam_ref, bet_ref, xe_ref)
    ya = _matT(xe, wfa_ref[...]) + bfc_ref[...][None, :]
    yb = _matT(xe, wfb_ref[...])
    cpad = jnp.zeros((NN, 126), F32)
    rpad = jnp.zeros((NP - NN, 128), F32)
    ya_out[...] = jnp.concatenate(
        [jnp.concatenate([ya, cpad], axis=1), rpad], axis=0)
    yb_out[...] = jnp.concatenate(
        [jnp.concatenate([yb, cpad], axis=1), rpad], axis=0)


_last_call = pl.pallas_call(
    _last_body,
    out_shape=(
        jax.ShapeDtypeStruct((NP, 128), F32),
        jax.ShapeDtypeStruct((NP, 128), F32),
    ),
)


def kernel(x, edge_index, edge_index_out, W_embed, b_embed, W_convs, b_convs,
           gammas, betas, W_fc, b_fc):
    srcf = edge_index[0]
    dst3 = edge_index[1].reshape(NW, NCH, CH)
    s2 = edge_index_out[0].reshape(NW, NCH, CH)
    d2 = edge_index_out[1].reshape(NW, NCH, CH)
    zz = jnp.zeros((NP, HH), F32)

    deg2 = _deg_call(dst3, zz)
    xe, h1 = _embed_call(x, W_embed, b_embed, W_convs[0])
    dinv, g = _scale_call(h1, deg2)
    for i in range(LL):
        agg = _agg_call(g, srcf, dst3, zz)
        if i < LL - 1:
            xe, g = _layer_call(agg, g, dinv, b_convs[i], gammas[i], betas[i],
                                xe, W_convs[i + 1])
        else:
            ya128, yb128 = _last_call(agg, g, dinv, b_convs[i], gammas[i],
                                      betas[i], xe, W_fc[:, :HH], W_fc[:, HH:],
                                      b_fc)
    return _fin_call(ya128, yb128, s2, d2)[:, :2]


# speedup vs baseline: 1.0572x; 1.0572x over previous
import jax
import jax.numpy as jnp
from jax import lax
from jax.experimental import pallas as pl
from jax.experimental.pallas import tpu as pltpu
from jax.experimental.pallas import tpu_sc as plsc

NN = 10000
EE = 320000
DD = 128
HH = 128
LL = 6
NC = 2
NS = 16
NW = NC * NS
EPT = EE // NW
CH = 80
NCH = EPT // CH
NP = 10240
F32 = jnp.float32

_MESH = plsc.VectorSubcoreMesh(core_axis_name="c", subcore_axis_name="s")


def _deg_body(dst3_hbm, zeros_hbm, deg_out, idx_v, ones_v, sem0, sem1,
              deg_sh):
    c = lax.axis_index("c")
    s = lax.axis_index("s")
    w = c * NS + s
    z16 = jnp.zeros((16,), F32)
    one_hot = jnp.where(lax.iota(jnp.int32, 16) == 0, 1.0, 0.0).astype(F32)

    @pl.when(s == 0)
    def _():
        pltpu.sync_copy(zeros_hbm, deg_sh)

    @pl.loop(0, CH)
    def _(r):
        for cc in range(1, 8):
            ones_v[r, pl.ds(cc * 16, 16)] = z16
        ones_v[r, pl.ds(0, 16)] = one_hot

    pltpu.sync_copy(dst3_hbm.at[w], idx_v)
    plsc.subcore_barrier()

    pltpu.async_copy(ones_v, deg_sh.at[idx_v.at[0]], sem0, add=True)
    pltpu.async_copy(ones_v, deg_sh.at[idx_v.at[1]], sem1, add=True)

    @pl.loop(2, NCH - 1, step=2)
    def _(j):
        pltpu.make_async_copy(ones_v, deg_sh.at[idx_v.at[j]], sem0).wait()
        pltpu.async_copy(ones_v, deg_sh.at[idx_v.at[j]], sem0, add=True)
        pltpu.make_async_copy(ones_v, deg_sh.at[idx_v.at[j + 1]], sem1).wait()
        pltpu.async_copy(ones_v, deg_sh.at[idx_v.at[j + 1]], sem1, add=True)

    pltpu.make_async_copy(ones_v, deg_sh.at[idx_v.at[0]], sem0).wait()
    pltpu.make_async_copy(ones_v, deg_sh.at[idx_v.at[1]], sem1).wait()
    pltpu.sync_copy(ones_v, deg_sh.at[idx_v.at[NCH - 1]], add=True)

    @pl.when(s == 0)
    def _():
        pltpu.sync_copy(deg_sh, deg_out.at[pl.ds(c * NP, NP)])


_deg_call = pl.kernel(
    _deg_body,
    out_type=jax.ShapeDtypeStruct((NC * NP, 128), F32),
    mesh=_MESH,
    scratch_types=[
        pltpu.VMEM((NCH, CH), jnp.int32),
        pltpu.VMEM((CH, 128), F32),
        pltpu.SemaphoreType.DMA,
        pltpu.SemaphoreType.DMA,
        pltpu.VMEM_SHARED((NP, 128), F32),
    ],
)


def _agg_body(g_hbm, srcf_hbm, dst3_hbm, zeros_hbm, agg_out, src_v, dst_v,
              buf0, buf1, semg0, semg1, sems0, sems1, agg_sh):
    c = lax.axis_index("c")
    s = lax.axis_index("s")
    w = c * NS + s

    @pl.when(s == 0)
    def _():
        pltpu.sync_copy(zeros_hbm, agg_sh)

    pltpu.sync_copy(srcf_hbm.at[pl.ds(w * EPT, EPT)], src_v)
    pltpu.sync_copy(dst3_hbm.at[w], dst_v)
    plsc.subcore_barrier()

    def _gat(j, buf, sem):
        return pltpu.async_copy(g_hbm.at[src_v.at[pl.ds(j * CH, CH)]], buf,
                                sem)

    _gat(0, buf0, semg0)
    _gat(1, buf1, semg1)

    @pl.loop(0, NCH - 3, step=2)
    def _(j):
        pltpu.make_async_copy(g_hbm.at[src_v.at[pl.ds(j * CH, CH)]], buf0,
                              semg0).wait()
        sa = pltpu.async_copy(buf0, agg_sh.at[dst_v.at[j]], sems0, add=True)
        pltpu.make_async_copy(g_hbm.at[src_v.at[pl.ds(j * CH, CH)]], buf1,
                              semg1).wait()
        sb = pltpu.async_copy(buf1, agg_sh.at[dst_v.at[j + 1]], sems1,
                              add=True)
        sa.wait()
        _gat(j + 2, buf0, semg0)
        sb.wait()
        _gat(j + 3, buf1, semg1)

    pltpu.make_async_copy(g_hbm.at[src_v.at[pl.ds(0, CH)]], buf0,
                          semg0).wait()
    pltpu.sync_copy(buf0, agg_sh.at[dst_v.at[NCH - 3]], add=True)
    pltpu.make_async_copy(g_hbm.at[src_v.at[pl.ds(0, CH)]], buf1,
                          semg1).wait()
    pltpu.sync_copy(buf1, agg_sh.at[dst_v.at[NCH - 2]], add=True)
    pltpu.async_copy(g_hbm.at[src_v.at[pl.ds((NCH - 1) * CH, CH)]], buf0,
                     semg0).wait()
    pltpu.sync_copy(buf0, agg_sh.at[dst_v.at[NCH - 1]], add=True)
    plsc.subcore_barrier()

    @pl.when(s == 0)
    def _():
        pltpu.sync_copy(agg_sh, agg_out.at[pl.ds(c * NP, NP)])


_agg_call = pl.kernel(
    _agg_body,
    out_type=jax.ShapeDtypeStruct((NC * NP, HH), F32),
    mesh=_MESH,
    scratch_types=[
        pltpu.VMEM((EPT,), jnp.int32),
        pltpu.VMEM((NCH, CH), jnp.int32),
        pltpu.VMEM((CH, HH), F32),
        pltpu.VMEM((CH, HH), F32),
        pltpu.SemaphoreType.DMA,
        pltpu.SemaphoreType.DMA,
        pltpu.SemaphoreType.DMA,
        pltpu.SemaphoreType.DMA,
        pltpu.VMEM_SHARED((NP, HH), F32),
    ],
)


def _fin_body(ya_hbm, yb_hbm, s2_hbm, d2_hbm, out_hbm, s2_v, d2_v, bufA0,
              bufB0, bufA1, bufB1, obuf0, obuf1, semA0, semB0, semA1, semB1):
    c = lax.axis_index("c")
    s = lax.axis_index("s")
    w = c * NS + s
    pltpu.sync_copy(s2_hbm.at[w], s2_v)
    pltpu.sync_copy(d2_hbm.at[w], d2_v)

    def _chunk(j, bufA, bufB, obuf):
        @pl.loop(0, CH, unroll=8)
        def _(r):
            obuf[r, :] = bufA[r, pl.ds(0, 16)] + bufB[r, pl.ds(0, 16)]

        pltpu.sync_copy(obuf, out_hbm.at[pl.ds(w * EPT + j * CH, CH)])

    def _gab(j, bufA, semA, bufB, semB):
        pltpu.async_copy(ya_hbm.at[s2_v.at[j]], bufA, semA)
        pltpu.async_copy(yb_hbm.at[d2_v.at[j]], bufB, semB)

    def _dab(bufA, semA, bufB, semB):
        pltpu.make_async_copy(ya_hbm.at[s2_v.at[0]], bufA, semA).wait()
        pltpu.make_async_copy(yb_hbm.at[d2_v.at[0]], bufB, semB).wait()

    _gab(0, bufA0, semA0, bufB0, semB0)
    _gab(1, bufA1, semA1, bufB1, semB1)

    @pl.loop(0, NCH - 3, step=2)
    def _(j):
        _dab(bufA0, semA0, bufB0, semB0)
        _chunk(j, bufA0, bufB0, obuf0)
        _gab(j + 2, bufA0, semA0, bufB0, semB0)
        _dab(bufA1, semA1, bufB1, semB1)
        _chunk(j + 1, bufA1, bufB1, obuf1)
        _gab(j + 3, bufA1, semA1, bufB1, semB1)

    _dab(bufA0, semA0, bufB0, semB0)
    _chunk(NCH - 3, bufA0, bufB0, obuf0)
    _gab(NCH - 1, bufA0, semA0, bufB0, semB0)
    _dab(bufA1, semA1, bufB1, semB1)
    _chunk(NCH - 2, bufA1, bufB1, obuf1)
    _dab(bufA0, semA0, bufB0, semB0)
    _chunk(NCH - 1, bufA0, bufB0, obuf0)


_fin_call = pl.kernel(
    _fin_body,
    out_type=jax.ShapeDtypeStruct((EE, 16), F32),
    mesh=_MESH,
    scratch_types=[
        pltpu.VMEM((NCH, CH), jnp.int32),
        pltpu.VMEM((NCH, CH), jnp.int32),
        pltpu.VMEM((CH, 128), F32),
        pltpu.VMEM((CH, 128), F32),
        pltpu.VMEM((CH, 128), F32),
        pltpu.VMEM((CH, 128), F32),
        pltpu.VMEM((CH, 16), F32),
        pltpu.VMEM((CH, 16), F32),
        pltpu.SemaphoreType.DMA,
        pltpu.SemaphoreType.DMA,
        pltpu.SemaphoreType.DMA,
        pltpu.SemaphoreType.DMA,
    ],
)


def _matT(a, b):
    return lax.dot_general(a, b, (((1,), (1,)), ((), ())),
                           preferred_element_type=F32)


def _embed_body(x_ref, we_ref, be_ref, w1_ref, xe_out, h1_out):
    xe = _matT(x_ref[...], we_ref[...]) + be_ref[...][None, :]
    xe_out[...] = xe
    h1_out[...] = _matT(xe, w1_ref[...])


_embed_call = pl.pallas_call(
    _embed_body,
    out_shape=(
        jax.ShapeDtypeStruct((NN, HH), F32),
        jax.ShapeDtypeStruct((NN, HH), F32),
    ),
)


def _scale_body(h1_ref, deg_ref, dinv_out, g_out):
    deg = deg_ref[0:NN, 0:1] + deg_ref[NP:NP + NN, 0:1] + 1.0
    dinv = lax.rsqrt(deg)
    dinv_out[...] = dinv
    g_out[...] = h1_ref[...] * dinv


_scale_call = pl.pallas_call(
    _scale_body,
    out_shape=(
        jax.ShapeDtypeStruct((NN, 1), F32),
        jax.ShapeDtypeStruct((NN, HH), F32),
    ),
)


def _layer_core(agg_ref, g_ref, dinv, b_ref, gam_ref, bet_ref, xe_ref):
    out = ((agg_ref[0:NN] + agg_ref[NP:NP + NN] + g_ref[...]) * dinv
           + b_ref[...][None, :])
    m = jnp.mean(out, axis=0, keepdims=True)
    d0 = out - m
    v = jnp.mean(d0 * d0, axis=0, keepdims=True)
    hh = d0 * lax.rsqrt(v + 1e-5) * gam_ref[...][None, :] + bet_ref[...][None, :]
    return xe_ref[...] + jnp.maximum(hh, 0.0)


def _layer_body(agg_ref, g_ref, dinv_ref, b_ref, gam_ref, bet_ref, xe_ref,
                wn_ref, xe_out, g_out):
    dinv = dinv_ref[...]
    xe = _layer_core(agg_ref, g_ref, dinv, b_ref, gam_ref, bet_ref, xe_ref)
    xe_out[...] = xe
    g_out[...] = _matT(xe, wn_ref[...]) * dinv


_layer_call = pl.pallas_call(
    _layer_body,
    out_shape=(
        jax.ShapeDtypeStruct((NN, HH), F32),
        jax.ShapeDtypeStruct((NN, HH), F32),
    ),
)


def _last_body(agg_ref, g_ref, dinv_ref, b_ref, gam_ref, bet_ref, xe_ref,
               wfa_ref, wfb_ref, bfc_ref, ya_out, yb_out):
    dinv = dinv_ref[...]
    xe = _layer_core(agg_ref, g_ref, dinv, b_ref, gam_ref, bet_ref, xe_ref)
    ya = _matT(xe, wfa_ref[...]) + bfc_ref[...][None, :]
    yb = _matT(xe, wfb_ref[...])
    cpad = jnp.zeros((NN, 126), F32)
    rpad = jnp.zeros((NP - NN, 128), F32)
    ya_out[...] = jnp.concatenate(
        [jnp.concatenate([ya, cpad], axis=1), rpad], axis=0)
    yb_out[...] = jnp.concatenate(
        [jnp.concatenate([yb, cpad], axis=1), rpad], axis=0)


_last_call = pl.pallas_call(
    _last_body,
    out_shape=(
        jax.ShapeDtypeStruct((NP, 128), F32),
        jax.ShapeDtypeStruct((NP, 128), F32),
    ),
)


def kernel(x, edge_index, edge_index_out, W_embed, b_embed, W_convs, b_convs,
           gammas, betas, W_fc, b_fc):
    srcf = edge_index[0]
    dst3 = edge_index[1].reshape(NW, NCH, CH)
    s2 = edge_index_out[0].reshape(NW, NCH, CH)
    d2 = edge_index_out[1].reshape(NW, NCH, CH)
    zz = jnp.zeros((NP, HH), F32)

    deg2 = _deg_call(dst3, zz)
    xe, h1 = _embed_call(x, W_embed, b_embed, W_convs[0])
    dinv, g = _scale_call(h1, deg2)
    for i in range(LL):
        agg = _agg_call(g, srcf, dst3, zz)
        if i < LL - 1:
            xe, g = _layer_call(agg, g, dinv, b_convs[i], gammas[i], betas[i],
                                xe, W_convs[i + 1])
        else:
            ya128, yb128 = _last_call(agg, g, dinv, b_convs[i], gammas[i],
                                      betas[i], xe, W_fc[:, :HH], W_fc[:, HH:],
                                      b_fc)
    return _fin_call(ya128, yb128, s2, d2)[:, :2]
